# baseline (speedup 1.0000x reference)
import functools

import jax
import jax.numpy as jnp
from jax import lax
from jax.experimental import pallas as pl
from jax.experimental.pallas import tpu as pltpu
from jax.experimental.pallas import tpu_sc as plsc

DIM = 64
BATCH = 16384
ROWS = 2 * BATCH
NC = 2
NS = 16
NW = NC * NS
BPW = ROWS // NW
RPW = BATCH // NW
NBLK = 125000
NGRP = BPW // 16


def _build_gather():
    mesh = plsc.VectorSubcoreMesh(core_axis_name="c", subcore_axis_name="s")

    @functools.partial(
        pl.kernel,
        mesh=mesh,
        compiler_params=pltpu.CompilerParams(needs_layout_passes=False),
        out_type=jax.ShapeDtypeStruct((BATCH, 2 * DIM), jnp.float32),
        scratch_types=[
            pltpu.VMEM((8, 128), jnp.int32),
            pltpu.VMEM((BPW,), jnp.int32),
            pltpu.VMEM((2, 16, 8, DIM), jnp.float32),
            pltpu.VMEM((RPW, 2 * DIM), jnp.float32),
            pltpu.SemaphoreType.DMA,
            pltpu.SemaphoreType.DMA,
        ],
    )
    def gather_k(idx_hbm, table_hbm, out_hbm,
                 idx_v, idx1_v, blk, out_v, sem0, sem1):
        wid = lax.axis_index("s") * NC + lax.axis_index("c")
        pltpu.sync_copy(idx_hbm.at[wid], idx_v)
        for su in range(8):
            for l in range(8):
                idx1_v[pl.ds(su * 128 + 16 * l, 16)] = idx_v[su, pl.ds(16 * l, 16)]

        sems = (sem0, sem1)

        def start_group(g, b):
            vv = idx1_v[pl.ds(g * 16, 16)]
            bb = lax.shift_right_logical(vv, 3)
            for j in range(16):
                pltpu.async_copy(table_hbm.at[bb[j]], blk.at[b, j], sems[b])

        def wait_group(g, b):
            vv = idx1_v[pl.ds(g * 16, 16)]
            bb = lax.shift_right_logical(vv, 3)
            for j in range(16):
                pltpu.make_async_copy(
                    table_hbm.at[bb[j]], blk.at[b, j], sems[b]).wait()

        def extract_group(g, b):
            vv = idx1_v[pl.ds(g * 16, 16)]
            sv = jnp.bitwise_and(vv, 7)
            r0 = g * 8
            for j in range(16):
                sub = sv[j]
                r = r0 + (j >> 1)
                c0 = 64 * (j & 1)
                for k in range(4):
                    out_v[r, pl.ds(c0 + 16 * k, 16)] = \
                        blk[b, j, sub, pl.ds(16 * k, 16)]

        start_group(0, 0)

        def body(s):
            for b in range(2):
                g = s + b

                @pl.when(g + 1 < NGRP)
                def _():
                    start_group(g + 1, 1 - b)

                wait_group(g, b)
                extract_group(g, b)

        pl.loop(0, NGRP, step=2)(body)
        pltpu.sync_copy(out_v, out_hbm.at[pl.ds(wid * RPW, RPW)])

    return gather_k


_gather = _build_gather()

BLK = 1024
GRID = BATCH // BLK


def _mlp_body(x_ref, w1t_ref, b1_ref, wn_ref, ws_ref, bias_ref,
              out_n_ref, out_s_ref):
    x = x_ref[...]
    h = jnp.dot(x, w1t_ref[...], preferred_element_type=jnp.float32)
    h = jnp.maximum(h + b1_ref[...], 0.0)
    n = jnp.sum(h * wn_ref[...], axis=1, keepdims=True) + bias_ref[0]
    s = jnp.sum(h * ws_ref[...], axis=1, keepdims=True) + bias_ref[1]
    out_n_ref[...] = jax.nn.sigmoid(n)
    out_s_ref[...] = jax.nn.sigmoid(s)


def _mlp(x, w1t, b1r, wn, ws, bias2):
    return pl.pallas_call(
        _mlp_body,
        grid=(GRID,),
        in_specs=[
            pl.BlockSpec((BLK, 2 * DIM), lambda i: (i, 0)),
            pl.BlockSpec((2 * DIM, 2 * DIM), lambda i: (0, 0)),
            pl.BlockSpec((1, 2 * DIM), lambda i: (0, 0)),
            pl.BlockSpec((1, 2 * DIM), lambda i: (0, 0)),
            pl.BlockSpec((1, 2 * DIM), lambda i: (0, 0)),
            pl.BlockSpec(memory_space=pltpu.SMEM),
        ],
        out_specs=[
            pl.BlockSpec((BLK, 1), lambda i: (i, 0)),
            pl.BlockSpec((BLK, 1), lambda i: (i, 0)),
        ],
        out_shape=[
            jax.ShapeDtypeStruct((BATCH, 1), jnp.float32),
            jax.ShapeDtypeStruct((BATCH, 1), jnp.float32),
        ],
    )(x, w1t, b1r, wn, ws, bias2)


def kernel(inputs, emb, W1, b1, Wn, bn, Ws, bs):
    idx = inputs.reshape(NW, 8, 128)
    table3 = emb.reshape(NBLK, 8, DIM)
    x = _gather(idx, table3)
    bias2 = jnp.concatenate([bn, bs])
    out_n, out_s = _mlp(x, W1.T, b1.reshape(1, 2 * DIM), Wn, Ws, bias2)
    return (out_n, out_s)
